# baseline (speedup 1.0000x reference)
import jax
import jax.numpy as jnp
from jax.experimental import pallas as pl
from jax.experimental.pallas import tpu as pltpu


_CIN1, _COUT1, _K1 = 3, 3, 5
_COUT2, _K2 = 5, 3
_H, _W = 32, 32
_OH1, _OW1 = _H - _K1 + 1, _W - _K1 + 1
_PH1, _PW1 = _OH1 // 2, _OW1 // 2
_OH2, _OW2 = _PH1 - _K2 + 1, _PW1 - _K2 + 1
_PH2, _PW2 = _OH2 // 2, _OW2 // 2
_NHID, _NOUT = 100, 10

_RB1 = 4
_NB1 = _OH1 // _RB1
_XR1 = _RB1 + _K1 - 1
_CHUNK1 = _XR1 * _W
_G1 = 64
_M1 = 2 * 2 * 2 * _G1
_F1L = _PH1 * _G1
_G2 = 256
_M2 = 2 * 2 * _G2


def _prep_operands(w1, b1, w2, b2, wf1, bf1, wf2, bf2):
    f32 = jnp.float32
    w1r = w1.astype(f32).reshape(_COUT1, _CIN1, _K1, _K1)
    w2r = w2.astype(f32).reshape(_COUT2, _CIN1, _K2, _K2)

    a1 = (jnp.arange(_XR1)[None, None, :, None]
          - (2 * jnp.arange(2)[None, :, None, None]
             + jnp.arange(2)[:, None, None, None])
          == jnp.arange(_K1)[None, None, None, :]).astype(f32)
    c1 = (jnp.arange(_W)[None, None, :, None]
          - (2 * jnp.arange(_PW1)[None, :, None, None]
             + jnp.arange(2)[:, None, None, None])
          == jnp.arange(_K1)[None, None, None, :]).astype(f32)
    band1 = jnp.einsum("oikl,pjhk,xqwl->xpjoqihw", w1r, a1, c1)
    band1 = band1.reshape(2 * 2 * 2, _COUT1 * _PW1, _CIN1, _XR1 * _W)
    band1 = jnp.pad(band1, ((0, 0), (0, _G1 - _COUT1 * _PW1), (0, 0), (0, 0)))
    band1t = band1.transpose(2, 3, 0, 1).reshape(_CIN1 * _CHUNK1, _M1)

    b1vec = jnp.broadcast_to(b1.astype(f32).reshape(1, _COUT1, 1),
                             (2, _COUT1, _PW1)).reshape(2, _COUT1 * _PW1)
    b1vec = jnp.pad(b1vec, ((0, 0), (0, _G1 - _COUT1 * _PW1))).reshape(1, 2 * _G1)

    a2 = (jnp.arange(_PH1)[None, None, :, None]
          - (2 * jnp.arange(_PH2)[None, :, None, None]
             + jnp.arange(2)[:, None, None, None])
          == jnp.arange(_K2)[None, None, None, :]).astype(f32)
    c2 = (jnp.arange(_PW1)[None, None, :, None]
          - (2 * jnp.arange(_PW2)[None, :, None, None]
             + jnp.arange(2)[:, None, None, None])
          == jnp.arange(_K2)[None, None, None, :]).astype(f32)
    band2 = jnp.einsum("oikl,pjhk,xsql->xpojshiq", w2r, a2, c2)
    band2 = band2.reshape(2 * 2, _COUT2 * _PH2 * _PW2, _PH1, _CIN1 * _PW1)
    band2 = jnp.pad(band2, ((0, 0), (0, _G2 - _COUT2 * _PH2 * _PW2),
                            (0, 0), (0, _G1 - _CIN1 * _PW1)))
    band2t = band2.transpose(2, 3, 0, 1).reshape(_F1L, _M2)

    b2vec = jnp.broadcast_to(b2.astype(f32).reshape(_COUT2, 1),
                             (_COUT2, _PH2 * _PW2)).reshape(1, -1)
    b2vec = jnp.pad(b2vec, ((0, 0), (0, _G2 - _COUT2 * _PH2 * _PW2)))

    wf1t = jnp.pad(wf1.astype(f32).T,
                   ((0, _G2 - _COUT2 * _PH2 * _PW2), (0, 0)))
    wf2t = wf2.astype(f32).T
    bf1v = bf1.astype(f32).reshape(1, _NHID)
    bf2v = bf2.astype(f32).reshape(1, _NOUT)
    return band1t, b1vec, band2t, b2vec, wf1t, bf1v, wf2t, bf2v


def _net_kernel(x_ref, wb1_ref, b1_ref, wb2_ref, b2_ref,
                wf1_ref, bf1_ref, wf2_ref, bf2_ref,
                o_ref, xbuf_ref, f1_ref, dma_sem):
    tn = xbuf_ref.shape[0]
    i = pl.program_id(0)
    cp = pltpu.make_async_copy(x_ref.at[pl.ds(i * tn, tn)], xbuf_ref, dma_sem)
    cp.start()
    cp.wait()
    xc = xbuf_ref[...].reshape(tn, _CIN1 * _H * _W)
    for blk in range(_NB1):
        z = None
        for ci in range(_CIN1):
            lo = ci * _H * _W + blk * _RB1 * _W
            xs = xc[:, lo:lo + _CHUNK1]
            wb = wb1_ref[ci * _CHUNK1:(ci + 1) * _CHUNK1, :]
            t = jnp.dot(xs, wb, preferred_element_type=jnp.float32)
            z = t if z is None else z + t
        pooled = jnp.maximum(jnp.maximum(z[:, 0:128], z[:, 128:256]),
                             jnp.maximum(z[:, 256:384], z[:, 384:512]))
        act = jnp.maximum(pooled + b1_ref[...], 0.0)
        f1_ref[:, 2 * blk * _G1:2 * blk * _G1 + 2 * _G1] = act

    z2 = jnp.dot(f1_ref[...], wb2_ref[...],
                 preferred_element_type=jnp.float32)
    pooled2 = jnp.maximum(jnp.maximum(z2[:, 0:256], z2[:, 256:512]),
                          jnp.maximum(z2[:, 512:768], z2[:, 768:1024]))
    feats = jnp.maximum(pooled2 + b2_ref[...], 0.0)

    h = jnp.dot(feats, wf1_ref[...], preferred_element_type=jnp.float32)
    h = jnp.maximum(h + bf1_ref[...], 0.0)
    o = jnp.dot(h, wf2_ref[...], preferred_element_type=jnp.float32)
    o_ref[...] = o + bf2_ref[...]


def kernel(x, w1, b1, w2, b2, wf1, bf1, wf2, bf2):
    ops = _prep_operands(w1, b1, w2, b2, wf1, bf1, wf2, bf2)
    band1t, b1vec, band2t, b2vec, wf1t, bf1v, wf2t, bf2v = ops

    n = x.shape[0]
    tile_n = n if n <= 256 else 256
    n_pad = ((n + tile_n - 1) // tile_n) * tile_n

    x_in = x.astype(jnp.float32)
    if n_pad != n:
        x_in = jnp.pad(x_in, ((0, n_pad - n), (0, 0), (0, 0), (0, 0)))

    out = pl.pallas_call(
        _net_kernel,
        out_shape=jax.ShapeDtypeStruct((n_pad, _NOUT), jnp.float32),
        grid=(n_pad // tile_n,),
        in_specs=[
            pl.BlockSpec(memory_space=pl.ANY),
            pl.BlockSpec((_CIN1 * _CHUNK1, _M1), lambda i: (0, 0)),
            pl.BlockSpec((1, 2 * _G1), lambda i: (0, 0)),
            pl.BlockSpec((_F1L, _M2), lambda i: (0, 0)),
            pl.BlockSpec((1, _G2), lambda i: (0, 0)),
            pl.BlockSpec((_G2, _NHID), lambda i: (0, 0)),
            pl.BlockSpec((1, _NHID), lambda i: (0, 0)),
            pl.BlockSpec((_NHID, _NOUT), lambda i: (0, 0)),
            pl.BlockSpec((1, _NOUT), lambda i: (0, 0)),
        ],
        out_specs=pl.BlockSpec((tile_n, _NOUT), lambda i: (i, 0)),
        scratch_shapes=[
            pltpu.VMEM((tile_n, _CIN1, _H, _W), jnp.float32),
            pltpu.VMEM((tile_n, _F1L), jnp.float32),
            pltpu.SemaphoreType.DMA,
        ],
        compiler_params=pltpu.CompilerParams(
            dimension_semantics=("parallel",),
            vmem_limit_bytes=48 * 1024 * 1024,
        ),
    )(x_in, band1t, b1vec, band2t, b2vec, wf1t, bf1v, wf2t, bf2v)

    return out[:n]


# speedup vs baseline: 6.9152x; 6.9152x over previous
import numpy as np

import jax
import jax.numpy as jnp
from jax.experimental import pallas as pl
from jax.experimental.pallas import tpu as pltpu


_CIN1, _COUT1, _K1 = 3, 3, 5
_COUT2, _K2 = 5, 3
_H, _W = 32, 32
_OH1, _OW1 = _H - _K1 + 1, _W - _K1 + 1
_PH1, _PW1 = _OH1 // 2, _OW1 // 2
_OH2, _OW2 = _PH1 - _K2 + 1, _PW1 - _K2 + 1
_PH2, _PW2 = _OH2 // 2, _OW2 // 2
_NFEAT = _COUT2 * _PH2 * _PW2
_NHID, _NOUT = 100, 10

_RB1 = 4
_NB1 = _OH1 // _RB1
_XR1 = _RB1 + _K1 - 1
_M1 = _COUT1 * _RB1 * _OW1
_KC1 = _XR1 * _W
_M2 = _COUT2 * _OH2 * _OW2
_KC2 = _PH1 * _PW1


def _mask1() -> np.ndarray:
    m = np.zeros((_K1 * _K1, 2, _RB1 // 2, _OW1, _XR1, _W), np.float32)
    for kh in range(_K1):
        for kw in range(_K1):
            for p in range(2):
                for j in range(_RB1 // 2):
                    for q in range(_OW1):
                        m[kh * _K1 + kw, p, j, q, 2 * j + p + kh, q + kw] = 1.0
    return m.reshape(_K1 * _K1, -1)


def _mask2() -> np.ndarray:
    m = np.zeros((_K2 * _K2, 2, _OH2 // 2, _OW2, _PH1, _PW1), np.float32)
    for kh in range(_K2):
        for kw in range(_K2):
            for p in range(2):
                for j in range(_OH2 // 2):
                    for q in range(_OW2):
                        m[kh * _K2 + kw, p, j, q, 2 * j + p + kh, q + kw] = 1.0
    return m.reshape(_K2 * _K2, -1)


_MASK1 = _mask1()
_MASK2 = _mask2()


def _band_matrices(w1, w2):
    w1r = w1.reshape(_COUT1, _CIN1, _K1 * _K1).transpose(1, 0, 2)
    band1 = jnp.dot(w1r.reshape(_CIN1 * _COUT1, _K1 * _K1), _MASK1,
                    precision=jax.lax.Precision.HIGHEST)
    band1 = band1.reshape(_CIN1, _M1, _KC1)
    w2r = w2.reshape(_COUT2, _COUT1, _K2 * _K2).transpose(1, 0, 2)
    band2 = jnp.dot(w2r.reshape(_COUT1 * _COUT2, _K2 * _K2), _MASK2,
                    precision=jax.lax.Precision.HIGHEST)
    band2 = band2.reshape(_COUT1, _M2, _KC2)
    return band1, band2


def _net_kernel(b1_ref, b2_ref,
                x_ref, wb1_ref, wb2_ref, wf1_ref, bf1_ref, wf2_ref, bf2_ref,
                o_ref, p1_ref):
    tn = x_ref.shape[-1]

    for blk in range(_NB1):
        z = None
        for ci in range(_CIN1):
            xs = x_ref[ci, _RB1 * blk:_RB1 * blk + _XR1, :, :].reshape(_KC1, tn)
            t = jnp.dot(wb1_ref[ci], xs, preferred_element_type=jnp.float32)
            z = t if z is None else z + t
        z = z.reshape(_COUT1, 2, (_RB1 // 2) * _OW1, tn)
        zh = jnp.maximum(z[:, 0], z[:, 1])
        zh = zh.reshape(_COUT1, _RB1 // 2, _PW1, 2, tn)
        zp = jnp.maximum(zh[:, :, :, 0], zh[:, :, :, 1])
        for co in range(_COUT1):
            p1_ref[co, 2 * blk:2 * blk + 2, :, :] = jnp.maximum(
                zp[co] + b1_ref[co], 0.0)

    z2 = None
    for ci in range(_COUT1):
        f1 = p1_ref[ci].reshape(_KC2, tn)
        t = jnp.dot(wb2_ref[ci], f1, preferred_element_type=jnp.float32)
        z2 = t if z2 is None else z2 + t
    z2 = z2.reshape(_COUT2, 2, _PH2 * _OW2, tn)
    zh2 = jnp.maximum(z2[:, 0], z2[:, 1])
    zh2 = zh2.reshape(_COUT2, _PH2, _PW2, 2, tn)
    zp2 = jnp.maximum(zh2[:, :, :, 0], zh2[:, :, :, 1])
    feats = [jnp.maximum(zp2[co] + b2_ref[co], 0.0).reshape(_PH2 * _PW2, tn)
             for co in range(_COUT2)]
    feats = jnp.concatenate(feats, axis=0)

    h = jnp.dot(wf1_ref[...], feats, preferred_element_type=jnp.float32)
    h = jnp.maximum(h + bf1_ref[...], 0.0)
    o = jnp.dot(wf2_ref[...], h, preferred_element_type=jnp.float32)
    o_ref[...] = o + bf2_ref[...]


def kernel(x, w1, b1, w2, b2, wf1, bf1, wf2, bf2):
    band1, band2 = _band_matrices(w1, w2)

    n = x.shape[0]
    tile_n = n if n <= 256 else 256
    n_pad = ((n + tile_n - 1) // tile_n) * tile_n

    x_t = jnp.transpose(x, (1, 2, 3, 0)).astype(jnp.float32)
    if n_pad != n:
        x_t = jnp.pad(x_t, ((0, 0), (0, 0), (0, 0), (0, n_pad - n)))

    out = pl.pallas_call(
        _net_kernel,
        out_shape=jax.ShapeDtypeStruct((_NOUT, n_pad), jnp.float32),
        grid=(n_pad // tile_n,),
        in_specs=[
            pl.BlockSpec(memory_space=pltpu.MemorySpace.SMEM),
            pl.BlockSpec(memory_space=pltpu.MemorySpace.SMEM),
            pl.BlockSpec((_CIN1, _H, _W, tile_n), lambda i: (0, 0, 0, i)),
            pl.BlockSpec((_CIN1, _M1, _KC1), lambda i: (0, 0, 0)),
            pl.BlockSpec((_COUT1, _M2, _KC2), lambda i: (0, 0, 0)),
            pl.BlockSpec((_NHID, _NFEAT), lambda i: (0, 0)),
            pl.BlockSpec((_NHID, 1), lambda i: (0, 0)),
            pl.BlockSpec((_NOUT, _NHID), lambda i: (0, 0)),
            pl.BlockSpec((_NOUT, 1), lambda i: (0, 0)),
        ],
        out_specs=pl.BlockSpec((_NOUT, tile_n), lambda i: (0, i)),
        scratch_shapes=[
            pltpu.VMEM((_CIN1, _PH1, _PW1, tile_n), jnp.float32),
        ],
        compiler_params=pltpu.CompilerParams(
            dimension_semantics=("parallel",),
            vmem_limit_bytes=48 * 1024 * 1024,
        ),
    )(b1, b2, x_t, band1, band2, wf1, bf1, wf2, bf2)

    return out[:, :n].T


# speedup vs baseline: 7.0443x; 1.0187x over previous
import numpy as np

import jax
import jax.numpy as jnp
from jax.experimental import pallas as pl
from jax.experimental.pallas import tpu as pltpu


_CIN1, _COUT1, _K1 = 3, 3, 5
_COUT2, _K2 = 5, 3
_H, _W = 32, 32
_OH1, _OW1 = _H - _K1 + 1, _W - _K1 + 1
_PH1, _PW1 = _OH1 // 2, _OW1 // 2
_OH2, _OW2 = _PH1 - _K2 + 1, _PW1 - _K2 + 1
_PH2, _PW2 = _OH2 // 2, _OW2 // 2
_NFEAT = _COUT2 * _PH2 * _PW2
_NHID, _NOUT = 100, 10

_RB1 = 4
_NB1 = _OH1 // _RB1
_XR1 = _RB1 + _K1 - 1
_M1 = _COUT1 * _RB1 * _OW1
_KC1 = _XR1 * _W
_M2 = _COUT2 * _OH2 * _OW2
_KC2 = _PH1 * _PW1


def _mask1() -> np.ndarray:
    m = np.zeros((_K1 * _K1, 2, _RB1 // 2, _OW1, _XR1, _W), np.float32)
    for kh in range(_K1):
        for kw in range(_K1):
            for p in range(2):
                for j in range(_RB1 // 2):
                    for q in range(_OW1):
                        m[kh * _K1 + kw, p, j, q, 2 * j + p + kh, q + kw] = 1.0
    return m.reshape(_K1 * _K1, -1)


def _mask2() -> np.ndarray:
    m = np.zeros((_K2 * _K2, 2, _OH2 // 2, _OW2, _PH1, _PW1), np.float32)
    for kh in range(_K2):
        for kw in range(_K2):
            for p in range(2):
                for j in range(_OH2 // 2):
                    for q in range(_OW2):
                        m[kh * _K2 + kw, p, j, q, 2 * j + p + kh, q + kw] = 1.0
    return m.reshape(_K2 * _K2, -1)


_MASK1 = _mask1()
_MASK2 = _mask2()


def _band_matrices(w1, w2):
    w1r = w1.reshape(_COUT1, _CIN1, _K1 * _K1).transpose(1, 0, 2)
    band1 = jnp.dot(w1r.reshape(_CIN1 * _COUT1, _K1 * _K1), _MASK1,
                    precision=jax.lax.Precision.HIGHEST)
    band1 = band1.reshape(_CIN1, _M1, _KC1)
    w2r = w2.reshape(_COUT2, _COUT1, _K2 * _K2).transpose(1, 0, 2)
    band2 = jnp.dot(w2r.reshape(_COUT1 * _COUT2, _K2 * _K2), _MASK2,
                    precision=jax.lax.Precision.HIGHEST)
    band2 = band2.reshape(_COUT1, _M2, _KC2)
    return band1, band2


def _net_kernel(b1_ref, b2_ref,
                x_ref, wb1_ref, wb2_ref, wf1_ref, bf1_ref, wf2_ref, bf2_ref,
                o_ref, p1_ref):
    tn = x_ref.shape[-1]

    for blk in range(_NB1):
        z = None
        for ci in range(_CIN1):
            xs = x_ref[ci, _RB1 * blk:_RB1 * blk + _XR1, :, :].reshape(_KC1, tn)
            t = jnp.dot(wb1_ref[ci], xs, preferred_element_type=jnp.float32)
            z = t if z is None else z + t
        z = z.reshape(_COUT1, 2, (_RB1 // 2) * _OW1, tn)
        zh = jnp.maximum(z[:, 0], z[:, 1])
        zh = zh.reshape(_COUT1, _RB1 // 2, _PW1, 2, tn)
        zp = jnp.maximum(zh[:, :, :, 0], zh[:, :, :, 1])
        for co in range(_COUT1):
            p1_ref[co, 2 * blk:2 * blk + 2, :, :] = jnp.maximum(
                zp[co] + b1_ref[co], 0.0)

    z2 = None
    for ci in range(_COUT1):
        f1 = p1_ref[ci].reshape(_KC2, tn)
        t = jnp.dot(wb2_ref[ci], f1, preferred_element_type=jnp.float32)
        z2 = t if z2 is None else z2 + t
    z2 = z2.reshape(_COUT2, 2, _PH2 * _OW2, tn)
    zh2 = jnp.maximum(z2[:, 0], z2[:, 1])
    zh2 = zh2.reshape(_COUT2, _PH2, _PW2, 2, tn)
    zp2 = jnp.maximum(zh2[:, :, :, 0], zh2[:, :, :, 1])
    feats = [jnp.maximum(zp2[co] + b2_ref[co], 0.0).reshape(_PH2 * _PW2, tn)
             for co in range(_COUT2)]
    feats = jnp.concatenate(feats, axis=0)

    h = jnp.dot(wf1_ref[...], feats, preferred_element_type=jnp.float32)
    h = jnp.maximum(h + bf1_ref[...], 0.0)
    o = jnp.dot(wf2_ref[...], h, preferred_element_type=jnp.float32)
    o_ref[...] = (o + bf2_ref[...]).T


def kernel(x, w1, b1, w2, b2, wf1, bf1, wf2, bf2):
    band1, band2 = _band_matrices(w1, w2)

    n = x.shape[0]
    tile_n = n if n <= 512 else 512
    n_pad = ((n + tile_n - 1) // tile_n) * tile_n

    x_t = jnp.transpose(x, (1, 2, 3, 0)).astype(jnp.float32)
    if n_pad != n:
        x_t = jnp.pad(x_t, ((0, 0), (0, 0), (0, 0), (0, n_pad - n)))

    out = pl.pallas_call(
        _net_kernel,
        out_shape=jax.ShapeDtypeStruct((n_pad, _NOUT), jnp.float32),
        grid=(n_pad // tile_n,),
        in_specs=[
            pl.BlockSpec(memory_space=pltpu.MemorySpace.SMEM),
            pl.BlockSpec(memory_space=pltpu.MemorySpace.SMEM),
            pl.BlockSpec((_CIN1, _H, _W, tile_n), lambda i: (0, 0, 0, i)),
            pl.BlockSpec((_CIN1, _M1, _KC1), lambda i: (0, 0, 0)),
            pl.BlockSpec((_COUT1, _M2, _KC2), lambda i: (0, 0, 0)),
            pl.BlockSpec((_NHID, _NFEAT), lambda i: (0, 0)),
            pl.BlockSpec((_NHID, 1), lambda i: (0, 0)),
            pl.BlockSpec((_NOUT, _NHID), lambda i: (0, 0)),
            pl.BlockSpec((_NOUT, 1), lambda i: (0, 0)),
        ],
        out_specs=pl.BlockSpec((tile_n, _NOUT), lambda i: (i, 0)),
        scratch_shapes=[
            pltpu.VMEM((_CIN1, _PH1, _PW1, tile_n), jnp.float32),
        ],
        compiler_params=pltpu.CompilerParams(
            dimension_semantics=("parallel",),
            vmem_limit_bytes=48 * 1024 * 1024,
        ),
    )(b1, b2, x_t, band1, band2, wf1, bf1, wf2, bf2)

    return out[:n]


# speedup vs baseline: 8.2846x; 1.1761x over previous
import numpy as np

import jax
import jax.numpy as jnp
from jax.experimental import pallas as pl
from jax.experimental.pallas import tpu as pltpu


_CIN1, _COUT1, _K1 = 3, 3, 5
_COUT2, _K2 = 5, 3
_H, _W = 32, 32
_OH1, _OW1 = _H - _K1 + 1, _W - _K1 + 1
_PH1, _PW1 = _OH1 // 2, _OW1 // 2
_OH2, _OW2 = _PH1 - _K2 + 1, _PW1 - _K2 + 1
_PH2, _PW2 = _OH2 // 2, _OW2 // 2
_NFEAT = _COUT2 * _PH2 * _PW2
_NHID, _NOUT = 100, 10

_RB1 = 4
_NB1 = _OH1 // _RB1
_XR1 = _RB1 + _K1 - 1
_M1 = _COUT1 * _RB1 * _OW1
_KC1 = _XR1 * _W
_PW1P = 16
_NB2 = _OH2 // 2
_M2 = _COUT2 * 2 * _OW2
_KC2 = _CIN1 * (_K2 + 1) * _PW1P


def _mask1() -> np.ndarray:
    m = np.zeros((_K1 * _K1, 2, _RB1 // 2, _OW1, _XR1, _W), np.float32)
    for kh in range(_K1):
        for kw in range(_K1):
            for p in range(2):
                for j in range(_RB1 // 2):
                    for q in range(_OW1):
                        m[kh * _K1 + kw, p, j, q, 2 * j + p + kh, q + kw] = 1.0
    return m.reshape(_K1 * _K1, -1)


def _mask2() -> np.ndarray:
    m = np.zeros((_CIN1 * _K2 * _K2, 2, _OW2, _CIN1, _K2 + 1, _PW1P),
                 np.float32)
    for ci in range(_CIN1):
        for kh in range(_K2):
            for kw in range(_K2):
                for p in range(2):
                    for q in range(_OW2):
                        m[(ci * _K2 + kh) * _K2 + kw, p, q,
                          ci, p + kh, q + kw] = 1.0
    return m.reshape(_CIN1 * _K2 * _K2, -1)


_MASK1 = _mask1()
_MASK2 = _mask2()


def _band_matrices(w1, w2):
    w1r = w1.reshape(_COUT1, _CIN1, _K1 * _K1).transpose(1, 0, 2)
    band1 = jnp.dot(w1r.reshape(_CIN1 * _COUT1, _K1 * _K1), _MASK1,
                    precision=jax.lax.Precision.HIGHEST)
    band1 = band1.reshape(_CIN1, _M1, _KC1)
    band2 = jnp.dot(w2.reshape(_COUT2, _CIN1 * _K2 * _K2), _MASK2,
                    precision=jax.lax.Precision.HIGHEST)
    band2 = band2.reshape(_M2, _KC2)
    return band1, band2


def _net_kernel(b1_ref, b2_ref,
                x_ref, wb1_ref, wb2_ref, wf1_ref, bf1_ref, wf2_ref, bf2_ref,
                o_ref, p1_ref, f2_ref):
    tn = x_ref.shape[-1]

    p1_ref[:, :, _PW1:_PW1P, :] = jnp.zeros((_CIN1, _PH1, _PW1P - _PW1, tn),
                                            jnp.float32)

    for blk in range(_NB1):
        z = None
        for ci in range(_CIN1):
            xs = x_ref[ci, _RB1 * blk:_RB1 * blk + _XR1, :, :].reshape(_KC1, tn)
            t = jnp.dot(wb1_ref[ci], xs, preferred_element_type=jnp.float32)
            z = t if z is None else z + t
        z = z.reshape(_COUT1, 2, (_RB1 // 2) * _OW1, tn)
        zh = jnp.maximum(z[:, 0], z[:, 1])
        zh = zh.reshape(_COUT1, _RB1 // 2, _PW1, 2, tn)
        zp = jnp.maximum(zh[:, :, :, 0], zh[:, :, :, 1])
        for co in range(_COUT1):
            p1_ref[co, 2 * blk:2 * blk + 2, 0:_PW1, :] = jnp.maximum(
                zp[co] + b1_ref[co], 0.0)

    for blk in range(_NB2):
        f1 = p1_ref[:, 2 * blk:2 * blk + _K2 + 1, :, :].reshape(_KC2, tn)
        z2 = jnp.dot(wb2_ref[...], f1, preferred_element_type=jnp.float32)
        z2 = z2.reshape(_COUT2, 2, _OW2, tn)
        zh2 = jnp.maximum(z2[:, 0], z2[:, 1])
        zh2 = zh2.reshape(_COUT2, _PW2, 2, tn)
        zp2 = jnp.maximum(zh2[:, :, 0], zh2[:, :, 1])
        for co in range(_COUT2):
            f2_ref[co, blk, :, :] = jnp.maximum(zp2[co] + b2_ref[co], 0.0)

    feats = f2_ref[...].reshape(_NFEAT, tn)

    h = jnp.dot(wf1_ref[...], feats, preferred_element_type=jnp.float32)
    h = jnp.maximum(h + bf1_ref[...], 0.0)
    o = jnp.dot(wf2_ref[...], h, preferred_element_type=jnp.float32)
    o_ref[...] = (o + bf2_ref[...]).T


def kernel(x, w1, b1, w2, b2, wf1, bf1, wf2, bf2):
    band1, band2 = _band_matrices(w1, w2)

    n = x.shape[0]
    tile_n = n if n <= 512 else 512
    n_pad = ((n + tile_n - 1) // tile_n) * tile_n

    x_t = jnp.transpose(x, (1, 2, 3, 0)).astype(jnp.float32)
    if n_pad != n:
        x_t = jnp.pad(x_t, ((0, 0), (0, 0), (0, 0), (0, n_pad - n)))

    out = pl.pallas_call(
        _net_kernel,
        out_shape=jax.ShapeDtypeStruct((n_pad, _NOUT), jnp.float32),
        grid=(n_pad // tile_n,),
        in_specs=[
            pl.BlockSpec(memory_space=pltpu.MemorySpace.SMEM),
            pl.BlockSpec(memory_space=pltpu.MemorySpace.SMEM),
            pl.BlockSpec((_CIN1, _H, _W, tile_n), lambda i: (0, 0, 0, i)),
            pl.BlockSpec((_CIN1, _M1, _KC1), lambda i: (0, 0, 0)),
            pl.BlockSpec((_M2, _KC2), lambda i: (0, 0)),
            pl.BlockSpec((_NHID, _NFEAT), lambda i: (0, 0)),
            pl.BlockSpec((_NHID, 1), lambda i: (0, 0)),
            pl.BlockSpec((_NOUT, _NHID), lambda i: (0, 0)),
            pl.BlockSpec((_NOUT, 1), lambda i: (0, 0)),
        ],
        out_specs=pl.BlockSpec((tile_n, _NOUT), lambda i: (i, 0)),
        scratch_shapes=[
            pltpu.VMEM((_CIN1, _PH1, _PW1P, tile_n), jnp.float32),
            pltpu.VMEM((_COUT2, _PH2, _PW2, tile_n), jnp.float32),
        ],
        compiler_params=pltpu.CompilerParams(
            dimension_semantics=("parallel",),
            vmem_limit_bytes=48 * 1024 * 1024,
        ),
    )(b1, b2, x_t, band1, band2, wf1, bf1, wf2, bf2)

    return out[:n]


# speedup vs baseline: 8.6253x; 1.0411x over previous
import numpy as np

import jax
import jax.numpy as jnp
from jax.experimental import pallas as pl
from jax.experimental.pallas import tpu as pltpu


_CIN1, _COUT1, _K1 = 3, 3, 5
_COUT2, _K2 = 5, 3
_H, _W = 32, 32
_OH1, _OW1 = _H - _K1 + 1, _W - _K1 + 1
_PH1, _PW1 = _OH1 // 2, _OW1 // 2
_OH2, _OW2 = _PH1 - _K2 + 1, _PW1 - _K2 + 1
_PH2, _PW2 = _OH2 // 2, _OW2 // 2
_NFEAT = _COUT2 * _PH2 * _PW2
_NHID, _NOUT = 100, 10

_RB1 = 4
_NB1 = _OH1 // _RB1
_XR1 = _RB1 + _K1 - 1
_MO1 = 2 * (_RB1 // 2) * _OW1
_M1 = _COUT1 * _MO1
_KC1 = _XR1 * _W
_PW1P = 16
_NB2 = _OH2 // 2
_MO2 = 2 * _OW2
_M2 = _COUT2 * _MO2
_KC2 = _CIN1 * (_K2 + 1) * _PW1P


def _mask1() -> np.ndarray:
    m = np.zeros((_K1 * _K1, 2, _RB1 // 2, _OW1, _XR1, _W), np.float32)
    for kh in range(_K1):
        for kw in range(_K1):
            for p in range(2):
                for j in range(_RB1 // 2):
                    for q in range(_OW1):
                        m[kh * _K1 + kw, p, j, q, 2 * j + p + kh, q + kw] = 1.0
    return m.reshape(_K1 * _K1, _MO1, _KC1)


def _mask2() -> np.ndarray:
    m = np.zeros((_CIN1 * _K2 * _K2, 2, _OW2, _CIN1, _K2 + 1, _PW1P),
                 np.float32)
    for ci in range(_CIN1):
        for kh in range(_K2):
            for kw in range(_K2):
                for p in range(2):
                    for q in range(_OW2):
                        m[(ci * _K2 + kh) * _K2 + kw, p, q,
                          ci, p + kh, q + kw] = 1.0
    return m.reshape(_CIN1 * _K2 * _K2, _MO2, _KC2)


_MASK1 = _mask1()
_MASK2 = _mask2()


def _net_kernel(w1_ref, b1_ref, w2_ref, b2_ref,
                x_ref, m1_ref, m2_ref, wf1_ref, bf1_ref, wf2_ref, bf2_ref,
                o_ref, wb1_ref, wb2_ref, p1_ref, f2_ref):
    tn = x_ref.shape[-1]

    @pl.when(pl.program_id(0) == 0)
    def _build():
        for ci in range(_CIN1):
            for co in range(_COUT1):
                acc = None
                for t in range(_K1 * _K1):
                    term = m1_ref[t] * w1_ref[(co * _CIN1 + ci) * _K1 * _K1 + t]
                    acc = term if acc is None else acc + term
                wb1_ref[ci, co * _MO1:(co + 1) * _MO1, :] = acc
        for co in range(_COUT2):
            acc = None
            for t in range(_CIN1 * _K2 * _K2):
                term = m2_ref[t] * w2_ref[co * _CIN1 * _K2 * _K2 + t]
                acc = term if acc is None else acc + term
            wb2_ref[co * _MO2:(co + 1) * _MO2, :] = acc
        p1_ref[:, :, _PW1:_PW1P, :] = jnp.zeros(
            (_CIN1, _PH1, _PW1P - _PW1, tn), jnp.float32)

    for blk in range(_NB1):
        z = None
        for ci in range(_CIN1):
            xs = x_ref[ci, _RB1 * blk:_RB1 * blk + _XR1, :, :].reshape(_KC1, tn)
            t = jnp.dot(wb1_ref[ci], xs, preferred_element_type=jnp.float32)
            z = t if z is None else z + t
        z = z.reshape(_COUT1, 2, (_RB1 // 2) * _OW1, tn)
        zh = jnp.maximum(z[:, 0], z[:, 1])
        zh = zh.reshape(_COUT1, _RB1 // 2, _PW1, 2, tn)
        zp = jnp.maximum(zh[:, :, :, 0], zh[:, :, :, 1])
        for co in range(_COUT1):
            p1_ref[co, 2 * blk:2 * blk + 2, 0:_PW1, :] = jnp.maximum(
                zp[co] + b1_ref[co], 0.0)

    for blk in range(_NB2):
        f1 = p1_ref[:, 2 * blk:2 * blk + _K2 + 1, :, :].reshape(_KC2, tn)
        z2 = jnp.dot(wb2_ref[...], f1, preferred_element_type=jnp.float32)
        z2 = z2.reshape(_COUT2, 2, _OW2, tn)
        zh2 = jnp.maximum(z2[:, 0], z2[:, 1])
        zh2 = zh2.reshape(_COUT2, _PW2, 2, tn)
        zp2 = jnp.maximum(zh2[:, :, 0], zh2[:, :, 1])
        for co in range(_COUT2):
            f2_ref[co, blk, :, :] = jnp.maximum(zp2[co] + b2_ref[co], 0.0)

    feats = f2_ref[...].reshape(_NFEAT, tn)

    h = jnp.dot(wf1_ref[...], feats, preferred_element_type=jnp.float32)
    h = jnp.maximum(h + bf1_ref[...], 0.0)
    o = jnp.dot(wf2_ref[...], h, preferred_element_type=jnp.float32)
    o_ref[...] = (o + bf2_ref[...]).T


def kernel(x, w1, b1, w2, b2, wf1, bf1, wf2, bf2):
    n = x.shape[0]
    tile_n = n if n <= 512 else 512
    n_pad = ((n + tile_n - 1) // tile_n) * tile_n

    x_t = jnp.transpose(x, (1, 2, 3, 0)).astype(jnp.float32)
    if n_pad != n:
        x_t = jnp.pad(x_t, ((0, 0), (0, 0), (0, 0), (0, n_pad - n)))

    out = pl.pallas_call(
        _net_kernel,
        out_shape=jax.ShapeDtypeStruct((n_pad, _NOUT), jnp.float32),
        grid=(n_pad // tile_n,),
        in_specs=[
            pl.BlockSpec(memory_space=pltpu.MemorySpace.SMEM),
            pl.BlockSpec(memory_space=pltpu.MemorySpace.SMEM),
            pl.BlockSpec(memory_space=pltpu.MemorySpace.SMEM),
            pl.BlockSpec(memory_space=pltpu.MemorySpace.SMEM),
            pl.BlockSpec((_CIN1, _H, _W, tile_n), lambda i: (0, 0, 0, i)),
            pl.BlockSpec((_K1 * _K1, _MO1, _KC1), lambda i: (0, 0, 0)),
            pl.BlockSpec((_CIN1 * _K2 * _K2, _MO2, _KC2), lambda i: (0, 0, 0)),
            pl.BlockSpec((_NHID, _NFEAT), lambda i: (0, 0)),
            pl.BlockSpec((_NHID, 1), lambda i: (0, 0)),
            pl.BlockSpec((_NOUT, _NHID), lambda i: (0, 0)),
            pl.BlockSpec((_NOUT, 1), lambda i: (0, 0)),
        ],
        out_specs=pl.BlockSpec((tile_n, _NOUT), lambda i: (i, 0)),
        scratch_shapes=[
            pltpu.VMEM((_CIN1, _M1, _KC1), jnp.float32),
            pltpu.VMEM((_M2, _KC2), jnp.float32),
            pltpu.VMEM((_CIN1, _PH1, _PW1P, tile_n), jnp.float32),
            pltpu.VMEM((_COUT2, _PH2, _PW2, tile_n), jnp.float32),
        ],
        compiler_params=pltpu.CompilerParams(
            dimension_semantics=("arbitrary",),
            vmem_limit_bytes=48 * 1024 * 1024,
        ),
    )(w1, b1, w2, b2, x_t, jnp.asarray(_MASK1), jnp.asarray(_MASK2),
      wf1, bf1, wf2, bf2)

    return out[:n]


# speedup vs baseline: 8.9826x; 1.0414x over previous
import numpy as np

import jax
import jax.numpy as jnp
from jax.experimental import pallas as pl
from jax.experimental.pallas import tpu as pltpu


_CIN1, _COUT1, _K1 = 3, 3, 5
_COUT2, _K2 = 5, 3
_H, _W = 32, 32
_OH1, _OW1 = _H - _K1 + 1, _W - _K1 + 1
_PH1, _PW1 = _OH1 // 2, _OW1 // 2
_OH2, _OW2 = _PH1 - _K2 + 1, _PW1 - _K2 + 1
_PH2, _PW2 = _OH2 // 2, _OW2 // 2
_NFEAT = _COUT2 * _PH2 * _PW2
_NHID, _NOUT = 100, 10

_RB1 = 4
_NB1 = _OH1 // _RB1
_XR1 = _RB1 + _K1 - 1
_MO1 = 2 * (_RB1 // 2) * _OW1
_M1 = _COUT1 * _MO1
_KC1 = _XR1 * _W
_PW1P = 16
_NB2 = _OH2 // 2
_MO2 = 2 * _OW2
_M2 = _COUT2 * _MO2
_KC2 = _CIN1 * (_K2 + 1) * _PW1P


def _mask1() -> np.ndarray:
    m = np.zeros((_K1 * _K1, 2, _RB1 // 2, _OW1, _XR1, _W), np.float32)
    for kh in range(_K1):
        for kw in range(_K1):
            for p in range(2):
                for j in range(_RB1 // 2):
                    for q in range(_OW1):
                        m[kh * _K1 + kw, p, j, q, 2 * j + p + kh, q + kw] = 1.0
    return m.reshape(_K1 * _K1, _MO1, _KC1)


def _mask2() -> np.ndarray:
    m = np.zeros((_CIN1 * _K2 * _K2, 2, _OW2, _CIN1, _K2 + 1, _PW1P),
                 np.float32)
    for ci in range(_CIN1):
        for kh in range(_K2):
            for kw in range(_K2):
                for p in range(2):
                    for q in range(_OW2):
                        m[(ci * _K2 + kh) * _K2 + kw, p, q,
                          ci, p + kh, q + kw] = 1.0
    return m.reshape(_CIN1 * _K2 * _K2, _MO2, _KC2)


_MASK1 = _mask1()
_MASK2 = _mask2()


def _net_kernel(w1_ref, b1_ref, w2_ref, b2_ref,
                x_ref, m1_ref, m2_ref, wf1_ref, bf1_ref, wf2_ref, bf2_ref,
                o_ref, wb1_ref, wb2_ref, p1_ref, f2_ref):
    tn = x_ref.shape[-1]

    @pl.when(pl.program_id(0) == 0)
    def _build():
        for ci in range(_CIN1):
            for co in range(_COUT1):
                acc = None
                for t in range(_K1 * _K1):
                    term = m1_ref[t] * w1_ref[(co * _CIN1 + ci) * _K1 * _K1 + t]
                    acc = term if acc is None else acc + term
                wb1_ref[ci, co * _MO1:(co + 1) * _MO1, :] = acc
        for co in range(_COUT2):
            acc = None
            for t in range(_CIN1 * _K2 * _K2):
                term = m2_ref[t] * w2_ref[co * _CIN1 * _K2 * _K2 + t]
                acc = term if acc is None else acc + term
            wb2_ref[co * _MO2:(co + 1) * _MO2, :] = acc
        p1_ref[:, :, _PW1:_PW1P, :] = jnp.zeros(
            (_CIN1, _PH1, _PW1P - _PW1, tn), jnp.float32)

    for blk in range(_NB1):
        z = None
        for ci in range(_CIN1):
            xs = x_ref[ci, _RB1 * blk:_RB1 * blk + _XR1, :, :].reshape(_KC1, tn)
            t = jnp.dot(wb1_ref[ci], xs, preferred_element_type=jnp.float32)
            z = t if z is None else z + t
        z = z.reshape(_COUT1, 2, (_RB1 // 2) * _OW1, tn)
        zh = jnp.maximum(z[:, 0], z[:, 1])
        zh = zh.reshape(_COUT1, _RB1 // 2, _PW1, 2, tn)
        zp = jnp.maximum(zh[:, :, :, 0], zh[:, :, :, 1])
        for co in range(_COUT1):
            p1_ref[co, 2 * blk:2 * blk + 2, 0:_PW1, :] = jnp.maximum(
                zp[co] + b1_ref[co], 0.0)

    for blk in range(_NB2):
        f1 = p1_ref[:, 2 * blk:2 * blk + _K2 + 1, :, :].reshape(_KC2, tn)
        z2 = jnp.dot(wb2_ref[...], f1, preferred_element_type=jnp.float32)
        z2 = z2.reshape(_COUT2, 2, _OW2, tn)
        zh2 = jnp.maximum(z2[:, 0], z2[:, 1])
        zh2 = zh2.reshape(_COUT2, _PW2, 2, tn)
        zp2 = jnp.maximum(zh2[:, :, 0], zh2[:, :, 1])
        for co in range(_COUT2):
            f2_ref[co, blk, :, :] = jnp.maximum(zp2[co] + b2_ref[co], 0.0)

    feats = f2_ref[...].reshape(_NFEAT, tn)

    h = jnp.dot(wf1_ref[...], feats, preferred_element_type=jnp.float32)
    h = jnp.maximum(h + bf1_ref[...], 0.0)
    o = jnp.dot(wf2_ref[...], h, preferred_element_type=jnp.float32)
    o_ref[...] = o + bf2_ref[...]


def kernel(x, w1, b1, w2, b2, wf1, bf1, wf2, bf2):
    n = x.shape[0]
    tile_n = n if n <= 512 else 512
    n_pad = ((n + tile_n - 1) // tile_n) * tile_n

    x_t = jnp.transpose(x, (1, 2, 3, 0)).astype(jnp.float32)
    if n_pad != n:
        x_t = jnp.pad(x_t, ((0, 0), (0, 0), (0, 0), (0, n_pad - n)))

    out = pl.pallas_call(
        _net_kernel,
        out_shape=jax.ShapeDtypeStruct((_NOUT, n_pad), jnp.float32),
        grid=(n_pad // tile_n,),
        in_specs=[
            pl.BlockSpec(memory_space=pltpu.MemorySpace.SMEM),
            pl.BlockSpec(memory_space=pltpu.MemorySpace.SMEM),
            pl.BlockSpec(memory_space=pltpu.MemorySpace.SMEM),
            pl.BlockSpec(memory_space=pltpu.MemorySpace.SMEM),
            pl.BlockSpec((_CIN1, _H, _W, tile_n), lambda i: (0, 0, 0, i)),
            pl.BlockSpec((_K1 * _K1, _MO1, _KC1), lambda i: (0, 0, 0)),
            pl.BlockSpec((_CIN1 * _K2 * _K2, _MO2, _KC2), lambda i: (0, 0, 0)),
            pl.BlockSpec((_NHID, _NFEAT), lambda i: (0, 0)),
            pl.BlockSpec((_NHID, 1), lambda i: (0, 0)),
            pl.BlockSpec((_NOUT, _NHID), lambda i: (0, 0)),
            pl.BlockSpec((_NOUT, 1), lambda i: (0, 0)),
        ],
        out_specs=pl.BlockSpec((_NOUT, tile_n), lambda i: (0, i)),
        scratch_shapes=[
            pltpu.VMEM((_CIN1, _M1, _KC1), jnp.float32),
            pltpu.VMEM((_M2, _KC2), jnp.float32),
            pltpu.VMEM((_CIN1, _PH1, _PW1P, tile_n), jnp.float32),
            pltpu.VMEM((_COUT2, _PH2, _PW2, tile_n), jnp.float32),
        ],
        compiler_params=pltpu.CompilerParams(
            dimension_semantics=("arbitrary",),
            vmem_limit_bytes=48 * 1024 * 1024,
        ),
    )(w1, b1, w2, b2, x_t, jnp.asarray(_MASK1), jnp.asarray(_MASK2),
      wf1, bf1, wf2, bf2)

    return out[:, :n].T
